# baseline (speedup 1.0000x reference)
import functools

import jax
import jax.numpy as jnp
from jax import lax
from jax.experimental import pallas as pl
from jax.experimental.pallas import tpu as pltpu
from jax.experimental.pallas import tpu_sc as plsc

_C = 0.95
_INV = 1.0 / 0.95
_R, _N = 4096, 8192
_NW = 32
_RPW = _R // _NW
_NG = _N // 16
_GC = 4
_NC = _NG // _GC
_U = 8
_ITERS = 13
_B = 4

_mesh = plsc.VectorSubcoreMesh(core_axis_name="c", subcore_axis_name="s")

_DNUMS = lax.GatherDimensionNumbers(
    offset_dims=(), collapsed_slice_dims=(0,), start_index_map=(0,))


def _perm(v, k):
    idx = jnp.bitwise_xor(lax.iota(jnp.int32, 16), k)
    return lax.gather(v, idx[:, None], _DNUMS, (1,),
                      mode=lax.GatherScatterMode.PROMISE_IN_BOUNDS)


def _hreduce(v, op):
    for k in (8, 4, 2, 1):
        v = op(v, _perm(v, k))
    return v[0]


def _div(num, den):
    bits = lax.bitcast_convert_type(den, jnp.int32)
    r = lax.bitcast_convert_type(jnp.int32(0x7EF127EA) - bits, jnp.float32)
    for _ in range(4):
        r = r * (2.0 - den * r)
    return num * r


@functools.partial(
    pl.kernel,
    out_type=jax.ShapeDtypeStruct((_R, _N), jnp.float32),
    mesh=_mesh,
    scratch_types=(
        [pltpu.VMEM((_N,), jnp.float32) for _ in range(2 * _B)]
        + [pltpu.VMEM((_NC * 16,), jnp.int32) for _ in range(_B)]
        + [pltpu.VMEM((_NC,), jnp.float32)]
        + [pltpu.VMEM((_NC * 16,), jnp.float32)]
        + [pltpu.SemaphoreType.DMA for _ in range(2 * _B)]
    ),
)
def _sparsegen_sc(x_hbm, out_hbm, *refs):
    row_bufs = refs[:_B]
    out_bufs = refs[_B:2 * _B]
    idx_bufs = refs[2 * _B:3 * _B]
    cmax_v = refs[3 * _B]
    cmc_v = refs[3 * _B + 1]
    in_sems = refs[3 * _B + 2:4 * _B + 2]
    out_sems = refs[4 * _B + 2:5 * _B + 2]

    cid = lax.axis_index("c")
    sid = lax.axis_index("s")
    wid = sid * 2 + cid
    base = wid * _RPW

    def zer(j, c0):
        for u in range(_U):
            for b in range(_B):
                out_bufs[b][pl.ds((j * _U + u) * 16, 16)] = (
                    jnp.zeros((16,), jnp.float32))
        return c0

    lax.fori_loop(0, _NG // _U, zer, 0)

    def solve_row(row_v, idx_v):
        def fused(oc, rm):
            vecs = []
            for t in range(16):
                base = (oc * 16 + t) * (_GC * 16)
                am = row_v[pl.ds(base, 16)]
                for u in range(1, _GC):
                    am = jnp.maximum(am, row_v[pl.ds(base + u * 16, 16)])
                vecs.append(am)
            k = 1
            lane = lax.iota(jnp.int32, 16)
            while len(vecs) > 1:
                nxt = []
                for i in range(len(vecs) // 2):
                    a, b = vecs[2 * i], vecs[2 * i + 1]
                    m1 = jnp.maximum(a, _perm(a, k))
                    m2 = jnp.maximum(b, _perm(b, k))
                    nxt.append(jnp.where((lane & k) == 0, m1, m2))
                vecs = nxt
                k *= 2
            cmax_v[pl.ds(oc * 16, 16)] = vecs[0]
            return jnp.maximum(rm, vecs[0])

        rm = lax.fori_loop(0, _NC // 16, fused,
                           jnp.full((16,), -3.0e38, jnp.float32))
        mx = _hreduce(rm, jnp.maximum)
        thr = mx - _C

        def bfind(oc, nc):
            cm = cmax_v[pl.ds(oc * 16, 16)]
            for t in range(16):
                jc = oc * 16 + t
                idx_v[pl.ds(nc * 16, 16)] = lax.broadcast(jc, (16,))
                cmc_v[pl.ds(nc * 16, 16)] = lax.broadcast(cm[t], (16,))
                nc = nc + jnp.where(cm[t] > thr, 1, 0).astype(jnp.int32)
            return nc

        nc = lax.fori_loop(0, _NC // 16, bfind, jnp.int32(0))

        def refilter(t, n_in):
            def rbody(i, n2):
                jcv = idx_v[pl.ds(i * 16, 16)]
                cmv = cmc_v[pl.ds(i * 16, 16)]
                idx_v[pl.ds(n2 * 16, 16)] = jcv
                cmc_v[pl.ds(n2 * 16, 16)] = cmv
                return n2 + jnp.where(cmv[0] > t, 1, 0).astype(jnp.int32)

            return lax.fori_loop(0, n_in, rbody, jnp.int32(0))

        def sk(tau, bound):
            def chunk(i, svv, kvv):
                jc = idx_v[pl.ds(i * 16, 16)][0]
                for u in range(_GC):
                    v = row_v[pl.ds((jc * _GC + u) * 16, 16)]
                    msk = v > tau
                    svv = svv + jnp.where(msk, v, 0.0)
                    kvv = kvv + jnp.where(msk, 1, 0).astype(jnp.int32)
                return svv, kvv

            def body2(p, c3):
                svv, kvv, svv2, kvv2 = c3
                svv, kvv = chunk(2 * p, svv, kvv)
                svv2, kvv2 = chunk(2 * p + 1, svv2, kvv2)
                return (svv, kvv, svv2, kvv2)

            z16f = jnp.zeros((16,), jnp.float32)
            z16i = jnp.zeros((16,), jnp.int32)
            svv, kvv, svv2, kvv2 = lax.fori_loop(
                0, bound // 2, body2, (z16f, z16i, z16f, z16i))

            def tailb(i, c3):
                svv, kvv = c3
                return chunk(bound - 1, svv, kvv)

            svv, kvv = lax.fori_loop(0, bound % 2, tailb, (svv, kvv))
            svv = svv + svv2
            kvv = kvv + kvv2
            return (_hreduce(svv, lambda a, b: a + b),
                    _hreduce(kvv, lambda a, b: a + b))

        s0, k0 = sk(thr, nc)

        def step(_, c4):
            lo, hi, s_lo, k_lo, done, ncc = c4
            t_m = _div(s_lo - _C, k_lo.astype(jnp.float32))
            t_b = 0.5 * (lo + hi)
            use_m = t_m >= t_b
            t = jnp.maximum(t_m, t_b)
            bound = jnp.where(done, 0, ncc)
            s, k = sk(t, bound)
            f = s - k.astype(jnp.float32) * t
            adv = f >= _C
            conv = jnp.logical_and(jnp.logical_and(use_m, adv), k == k_lo)
            lo2 = jnp.where(adv, t, lo)
            hi2 = jnp.where(adv, hi, t)
            s2 = jnp.where(adv, s, s_lo)
            k2 = jnp.where(adv, k, k_lo)
            skip_rf = jnp.logical_or(done, jnp.logical_not(adv))
            ncr = refilter(t, jnp.where(skip_rf, 0, ncc))
            nc2 = jnp.where(skip_rf, ncc, ncr)
            return (jnp.where(done, lo, lo2),
                    jnp.where(done, hi, hi2),
                    jnp.where(done, s_lo, s2),
                    jnp.where(done, k_lo, k2),
                    jnp.logical_or(done, conv),
                    nc2)

        lo, hi, s_lo, k_lo, done, ncf = lax.fori_loop(
            0, _ITERS, step, (thr, mx, s0, k0, jnp.bool_(False), nc))
        tau = _div(s_lo - _C, k_lo.astype(jnp.float32))
        return tau, ncf

    def write_out(row_v, out_v, idx_v, tau, nc):
        def body(i, c5):
            jc = idx_v[pl.ds(i * 16, 16)][0]
            for u in range(_GC):
                sl = pl.ds((jc * _GC + u) * 16, 16)
                out_v[sl] = jnp.maximum(row_v[sl] - tau, 0.0) * _INV
            return c5

        lax.fori_loop(0, nc, body, 0)

    def rezero(out_v, idx_v, nc):
        def body(i, c6):
            jc = idx_v[pl.ds(i * 16, 16)][0]
            for u in range(_GC):
                out_v[pl.ds((jc * _GC + u) * 16, 16)] = (
                    jnp.zeros((16,), jnp.float32))
            return c6

        lax.fori_loop(0, nc, body, 0)

    def in_copy(i, b):
        r = jnp.minimum(base + i * _B + b, _R - 1)
        return pltpu.make_async_copy(x_hbm.at[r], row_bufs[b], in_sems[b])

    for b in range(_B):
        in_copy(jnp.int32(0), b).start()

    def do_block(i, carry):
        r0 = base + i * _B
        out_cps = [pltpu.make_async_copy(out_bufs[b], out_hbm.at[r0 + b],
                                         out_sems[b]) for b in range(_B)]
        taus = []
        for b in range(_B):
            in_copy(i, b).wait()
            tau, nc = solve_row(row_bufs[b], idx_bufs[b])
            write_out(row_bufs[b], out_bufs[b], idx_bufs[b], tau, nc)
            out_cps[b].start()
            in_copy(i + 1, b).start()
            taus.append(nc)
        for b in range(_B):
            out_cps[b].wait()
            rezero(out_bufs[b], idx_bufs[b], taus[b])
        return carry

    lax.fori_loop(0, _RPW // _B, do_block, 0)
    for b in range(_B):
        in_copy(jnp.int32(_RPW // _B), b).wait()


def kernel(input):
    x = input.reshape(_R, _N)
    out = _sparsegen_sc(x)
    return out.reshape(input.shape)


# speedup vs baseline: 1.1098x; 1.1098x over previous
import functools

import jax
import jax.numpy as jnp
from jax import lax
from jax.experimental import pallas as pl
from jax.experimental.pallas import tpu as pltpu
from jax.experimental.pallas import tpu_sc as plsc

_C = 0.95
_INV = 1.0 / 0.95
_R, _N = 4096, 8192
_NW = 32
_RPW = _R // _NW
_NG = _N // 16
_GC = 4
_NC = _NG // _GC
_U = 8
_ITERS = 13
_B = 4

_mesh = plsc.VectorSubcoreMesh(core_axis_name="c", subcore_axis_name="s")

_DNUMS = lax.GatherDimensionNumbers(
    offset_dims=(), collapsed_slice_dims=(0,), start_index_map=(0,))


def _perm(v, k):
    idx = jnp.bitwise_xor(lax.iota(jnp.int32, 16), k)
    return lax.gather(v, idx[:, None], _DNUMS, (1,),
                      mode=lax.GatherScatterMode.PROMISE_IN_BOUNDS)


def _hreduce(v, op):
    for k in (8, 4, 2, 1):
        v = op(v, _perm(v, k))
    return v[0]


def _div(num, den):
    bits = lax.bitcast_convert_type(den, jnp.int32)
    r = lax.bitcast_convert_type(jnp.int32(0x7EF127EA) - bits, jnp.float32)
    for _ in range(4):
        r = r * (2.0 - den * r)
    return num * r


@functools.partial(
    pl.kernel,
    out_type=jax.ShapeDtypeStruct((_R, _N), jnp.float32),
    mesh=_mesh,
    scratch_types=(
        [pltpu.VMEM((_N,), jnp.float32) for _ in range(2 * _B)]
        + [pltpu.VMEM((_NC * 16,), jnp.int32) for _ in range(_B)]
        + [pltpu.VMEM((_NC,), jnp.float32)]
        + [pltpu.VMEM((_NC * 16,), jnp.float32)]
        + [pltpu.SemaphoreType.DMA for _ in range(2 * _B)]
    ),
)
def _sparsegen_sc(x_hbm, out_hbm, *refs):
    row_bufs = refs[:_B]
    out_bufs = refs[_B:2 * _B]
    idx_bufs = refs[2 * _B:3 * _B]
    cmax_v = refs[3 * _B]
    cmc_v = refs[3 * _B + 1]
    in_sems = refs[3 * _B + 2:4 * _B + 2]
    out_sems = refs[4 * _B + 2:5 * _B + 2]

    cid = lax.axis_index("c")
    sid = lax.axis_index("s")
    wid = sid * 2 + cid
    base = wid * _RPW

    def zer(j, c0):
        for u in range(_U):
            for b in range(_B):
                out_bufs[b][pl.ds((j * _U + u) * 16, 16)] = (
                    jnp.zeros((16,), jnp.float32))
        return c0

    lax.fori_loop(0, _NG // _U, zer, 0)

    def solve_row(row_v, idx_v):
        def fused(oc, rm):
            vecs = []
            for t in range(16):
                base = (oc * 16 + t) * (_GC * 16)
                am = row_v[pl.ds(base, 16)]
                for u in range(1, _GC):
                    am = jnp.maximum(am, row_v[pl.ds(base + u * 16, 16)])
                vecs.append(am)
            k = 1
            lane = lax.iota(jnp.int32, 16)
            while len(vecs) > 1:
                nxt = []
                for i in range(len(vecs) // 2):
                    a, b = vecs[2 * i], vecs[2 * i + 1]
                    m1 = jnp.maximum(a, _perm(a, k))
                    m2 = jnp.maximum(b, _perm(b, k))
                    nxt.append(jnp.where((lane & k) == 0, m1, m2))
                vecs = nxt
                k *= 2
            cmax_v[pl.ds(oc * 16, 16)] = vecs[0]
            return jnp.maximum(rm, vecs[0])

        rm = lax.fori_loop(0, _NC // 16, fused,
                           jnp.full((16,), -3.0e38, jnp.float32))
        mx = _hreduce(rm, jnp.maximum)
        thr = mx - _C

        def bfind(oc, nc):
            cm = cmax_v[pl.ds(oc * 16, 16)]
            for t in range(16):
                jc = oc * 16 + t
                idx_v[pl.ds(nc * 16, 16)] = lax.broadcast(jc, (16,))
                cmc_v[pl.ds(nc * 16, 16)] = lax.broadcast(cm[t], (16,))
                nc = nc + jnp.where(cm[t] > thr, 1, 0).astype(jnp.int32)
            return nc

        nc = lax.fori_loop(0, _NC // 16, bfind, jnp.int32(0))

        def refilter(t, n_in):
            def rbody(i, n2):
                jcv = idx_v[pl.ds(i * 16, 16)]
                cmv = cmc_v[pl.ds(i * 16, 16)]
                idx_v[pl.ds(n2 * 16, 16)] = jcv
                cmc_v[pl.ds(n2 * 16, 16)] = cmv
                return n2 + jnp.where(cmv[0] > t, 1, 0).astype(jnp.int32)

            return lax.fori_loop(0, n_in, rbody, jnp.int32(0))

        def sk(tau, bound):
            def body(i, c3):
                svv, kvv = c3
                jc = idx_v[pl.ds(i * 16, 16)][0]
                for u in range(_GC):
                    v = row_v[pl.ds((jc * _GC + u) * 16, 16)]
                    msk = v > tau
                    svv = svv + jnp.where(msk, v, 0.0)
                    kvv = kvv + jnp.where(msk, 1, 0).astype(jnp.int32)
                return (svv, kvv)

            svv, kvv = lax.fori_loop(
                0, bound, body,
                (jnp.zeros((16,), jnp.float32), jnp.zeros((16,), jnp.int32)))
            return (_hreduce(svv, lambda a, b: a + b),
                    _hreduce(kvv, lambda a, b: a + b))

        s0, k0 = sk(thr, nc)

        def step(_, c4):
            lo, hi, s_lo, k_lo, done, ncc = c4
            t_m = _div(s_lo - _C, k_lo.astype(jnp.float32))
            t_b = 0.5 * (lo + hi)
            use_m = t_m >= t_b
            t = jnp.maximum(t_m, t_b)
            bound = jnp.where(done, 0, ncc)
            s, k = sk(t, bound)
            f = s - k.astype(jnp.float32) * t
            adv = f >= _C
            conv = jnp.logical_and(jnp.logical_and(use_m, adv), k == k_lo)
            lo2 = jnp.where(adv, t, lo)
            hi2 = jnp.where(adv, hi, t)
            s2 = jnp.where(adv, s, s_lo)
            k2 = jnp.where(adv, k, k_lo)
            skip_rf = jnp.logical_or(done, jnp.logical_not(adv))
            ncr = refilter(t, jnp.where(skip_rf, 0, ncc))
            nc2 = jnp.where(skip_rf, ncc, ncr)
            return (jnp.where(done, lo, lo2),
                    jnp.where(done, hi, hi2),
                    jnp.where(done, s_lo, s2),
                    jnp.where(done, k_lo, k2),
                    jnp.logical_or(done, conv),
                    nc2)

        lo, hi, s_lo, k_lo, done, ncf = lax.fori_loop(
            0, _ITERS, step, (thr, mx, s0, k0, jnp.bool_(False), nc))
        tau = _div(s_lo - _C, k_lo.astype(jnp.float32))
        return tau, ncf

    def write_out(row_v, out_v, idx_v, tau, nc):
        def body(i, c5):
            jc = idx_v[pl.ds(i * 16, 16)][0]
            for u in range(_GC):
                sl = pl.ds((jc * _GC + u) * 16, 16)
                out_v[sl] = jnp.maximum(row_v[sl] - tau, 0.0) * _INV
            return c5

        lax.fori_loop(0, nc, body, 0)

    def rezero(out_v, idx_v, nc):
        def body(i, c6):
            jc = idx_v[pl.ds(i * 16, 16)][0]
            for u in range(_GC):
                out_v[pl.ds((jc * _GC + u) * 16, 16)] = (
                    jnp.zeros((16,), jnp.float32))
            return c6

        lax.fori_loop(0, nc, body, 0)

    def in_copy(i, b):
        r = jnp.minimum(base + i * _B + b, _R - 1)
        return pltpu.make_async_copy(x_hbm.at[r], row_bufs[b], in_sems[b])

    for b in range(_B):
        in_copy(jnp.int32(0), b).start()

    def do_block(i, carry):
        r0 = base + i * _B
        out_cps = [pltpu.make_async_copy(out_bufs[b], out_hbm.at[r0 + b],
                                         out_sems[b]) for b in range(_B)]
        taus = []
        for b in range(_B):
            in_copy(i, b).wait()
            tau, nc = solve_row(row_bufs[b], idx_bufs[b])
            write_out(row_bufs[b], out_bufs[b], idx_bufs[b], tau, nc)
            out_cps[b].start()
            in_copy(i + 1, b).start()
            taus.append(nc)
        for b in range(_B):
            out_cps[b].wait()
            rezero(out_bufs[b], idx_bufs[b], taus[b])
        return carry

    lax.fori_loop(0, _RPW // _B, do_block, 0)
    for b in range(_B):
        in_copy(jnp.int32(_RPW // _B), b).wait()


def kernel(input):
    x = input.reshape(_R, _N)
    out = _sparsegen_sc(x)
    return out.reshape(input.shape)
